# baseline (speedup 1.0000x reference)
import jax
import jax.numpy as jnp
from jax.experimental import pallas as pl
from jax.experimental.pallas import tpu as pltpu

_P = 7


def _argmax_planes(x_ref, o_ref, val_ref, idx_ref):
    k = pl.program_id(0)
    nk = pl.num_programs(0)

    @pl.when(k == 0)
    def _():
        val_ref[...] = jnp.full(val_ref.shape, -jnp.inf, jnp.float32)
        idx_ref[...] = jnp.zeros(idx_ref.shape, jnp.int32)

    base = k * _P
    s = x_ref.shape[2]
    tl = 256
    for c in range(s // tl):
        sl = pl.ds(c * tl, tl)
        val = val_ref[:, sl]
        idx = idx_ref[:, sl]
        for p in range(_P):
            xp = x_ref[p, :, sl]
            better = xp > val
            val = jnp.where(better, xp, val)
            idx = jnp.where(better, base + p, idx)
        val_ref[:, sl] = val
        idx_ref[:, sl] = idx

    @pl.when(k == nk - 1)
    def _():
        o_ref[...] = idx_ref[...]


def kernel(input_0):
    b, s, n = input_0.shape
    assert n % _P == 0
    xt = jnp.transpose(input_0, (2, 0, 1))
    out = pl.pallas_call(
        _argmax_planes,
        grid=(n // _P,),
        in_specs=[pl.BlockSpec((_P, b, s), lambda k: (k, 0, 0))],
        out_specs=pl.BlockSpec((b, s), lambda k: (0, 0)),
        out_shape=jax.ShapeDtypeStruct((b, s), jnp.int32),
        scratch_shapes=[
            pltpu.VMEM((b, s), jnp.float32),
            pltpu.VMEM((b, s), jnp.int32),
        ],
        compiler_params=pltpu.CompilerParams(
            dimension_semantics=("arbitrary",)
        ),
    )(xt)
    return out.astype(jnp.int64)
